# Initial kernel scaffold; baseline (speedup 1.0000x reference)
import jax
import jax.numpy as jnp
from jax.experimental import pallas as pl


def kernel(edge_index, h):
    raise NotImplementedError("write your pallas kernel here")


# speedup vs baseline: 3.1742x; 3.1742x over previous
import functools

import jax
import jax.numpy as jnp
from jax import lax
from jax.experimental import pallas as pl
from jax.experimental.pallas import tpu as pltpu
from jax.experimental.pallas import tpu_sc as plsc

N_NODES = 10000
N_EDGES = 320000
D = 128
NC = 2
NS = 16
NW = NC * NS
EPW = N_EDGES // NW
C = 80
NCHUNK = EPW // C
ROWS_PER_TILE = 624


def _lane_shuffle(x, perm):
    dnums = lax.GatherDimensionNumbers(
        offset_dims=(), collapsed_slice_dims=(0,), start_index_map=(0,))
    return lax.gather(x, perm[:, None], dnums, slice_sizes=(1,),
                      mode=lax.GatherScatterMode.PROMISE_IN_BOUNDS)


def _body(src_hbm, dst_hbm, h_hbm, out_hbm,
          h_sh, sidx, didx, rows_s, rows_d, out_v, sem_s, sem_d):
    c = lax.axis_index("c")
    s = lax.axis_index("s")
    wid = s * NC + c

    r0 = s * ROWS_PER_TILE
    pltpu.sync_copy(h_hbm.at[pl.ds(r0, ROWS_PER_TILE)],
                    h_sh.at[pl.ds(r0, ROWS_PER_TILE)])
    tail = NS * ROWS_PER_TILE
    @pl.when(s == 0)
    def _():
        pltpu.sync_copy(h_hbm.at[pl.ds(tail, N_NODES - tail)],
                        h_sh.at[pl.ds(tail, N_NODES - tail)])
    plsc.subcore_barrier()

    def chunk(k, _):
        base = wid * EPW + k * C
        pltpu.sync_copy(src_hbm.at[pl.ds(base, C)], sidx)
        pltpu.sync_copy(dst_hbm.at[pl.ds(base, C)], didx)
        cp_s = pltpu.async_copy(h_sh.at[sidx], rows_s, sem_s)
        cp_d = pltpu.async_copy(h_sh.at[didx], rows_d, sem_d)
        cp_s.wait()
        cp_d.wait()

        lane = lax.iota(jnp.int32, 16)
        perms = [lane ^ sh for sh in (1, 2, 4, 8)]

        def group(g, _):
            vec = jnp.zeros((16,), jnp.float32)
            for i in range(16):
                e = g * 16 + i
                acc = rows_s[e, pl.ds(0, 16)] * rows_d[e, pl.ds(0, 16)]
                for j in range(1, D // 16):
                    acc = acc + rows_s[e, pl.ds(j * 16, 16)] * rows_d[e, pl.ds(j * 16, 16)]
                for p in perms:
                    acc = acc + _lane_shuffle(acc, p)
                vec = jnp.where(lane == i, acc, vec)
            out_v[pl.ds(g * 16, 16)] = vec
            return 0

        lax.fori_loop(0, C // 16, group, 0)
        pltpu.sync_copy(out_v, out_hbm.at[pl.ds(base, C)])
        return 0

    lax.fori_loop(0, NCHUNK, chunk, 0)


@jax.jit
def _scores(src, dst, h):
    mesh = plsc.VectorSubcoreMesh(core_axis_name="c", subcore_axis_name="s")
    return pl.kernel(
        _body,
        out_type=jax.ShapeDtypeStruct((N_EDGES,), jnp.float32),
        mesh=mesh,
        scratch_types=[
            pltpu.VMEM_SHARED((N_NODES, D), jnp.float32),
            pltpu.VMEM((C,), jnp.int32),
            pltpu.VMEM((C,), jnp.int32),
            pltpu.VMEM((C, D), jnp.float32),
            pltpu.VMEM((C, D), jnp.float32),
            pltpu.VMEM((C,), jnp.float32),
            pltpu.SemaphoreType.DMA,
            pltpu.SemaphoreType.DMA,
        ],
    )(src, dst, h)


def kernel(edge_index, h):
    ei = edge_index.astype(jnp.int32)
    scores = _scores(ei[0], ei[1], h)
    return scores.reshape(N_EDGES, 1)
